# Initial kernel scaffold; baseline (speedup 1.0000x reference)
import jax
import jax.numpy as jnp
from jax.experimental import pallas as pl


def kernel(x, edge_index, W_self, W_neigh, bias):
    raise NotImplementedError("write your pallas kernel here")


# speedup vs baseline: 5.9745x; 5.9745x over previous
import functools

import jax
import jax.numpy as jnp
from jax import lax
from jax.experimental import pallas as pl
from jax.experimental.pallas import tpu as pltpu
from jax.experimental.pallas import tpu_sc as plsc

N = 10000
D = 128
HD = D // 2
E = 320000
NC = 2
NS = 16
EPT = E // NS
CHUNK = 80
NCHUNK = EPT // CHUNK
ACC_N = 10240
RPT = ACC_N // NS
ZROWS = 128
DEGW = 16


def _sc_aggregate(x2, row_idx, col_idx):
    mesh = plsc.VectorSubcoreMesh(core_axis_name="c", subcore_axis_name="s")

    @functools.partial(
        pl.kernel,
        out_type=(
            jax.ShapeDtypeStruct((NC * ACC_N, HD), jnp.float32),
            jax.ShapeDtypeStruct((NC * ACC_N, DEGW), jnp.float32),
        ),
        mesh=mesh,
        scratch_types=[
            pltpu.VMEM((NCHUNK, CHUNK), jnp.int32),
            pltpu.VMEM((NCHUNK, CHUNK), jnp.int32),
            pltpu.VMEM((NCHUNK, CHUNK), jnp.int32),
            pltpu.VMEM((CHUNK, HD), jnp.float32),
            pltpu.VMEM((CHUNK, DEGW), jnp.float32),
            pltpu.VMEM((ZROWS, HD), jnp.float32),
            pltpu.VMEM((ZROWS, DEGW), jnp.float32),
            pltpu.VMEM_SHARED((ACC_N, HD), jnp.float32),
            pltpu.VMEM_SHARED((ACC_N, DEGW), jnp.float32),
        ],
        compiler_params=pltpu.CompilerParams(use_tc_tiling_on_sc=False),
    )
    def sc_kernel(x_hbm, row_hbm, col_hbm, part_hbm, degp_hbm,
                  row_v, rowx_v, col_v, gbuf, ones_v, zrow, zdeg,
                  acc_s, deg_s):
        c = lax.axis_index("c")
        s = lax.axis_index("s")

        zeros16 = jnp.zeros((16,), jnp.float32)
        ones16 = jnp.ones((16,), jnp.float32)

        @pl.loop(0, ZROWS)
        def _(i):
            zdeg[i, :] = zeros16

            @pl.loop(0, HD // 16)
            def _(k):
                zrow[i, pl.ds(k * 16, 16)] = zeros16

        @pl.loop(0, CHUNK)
        def _(i):
            ones_v[i, :] = ones16

        @pl.loop(0, RPT // ZROWS)
        def _(q):
            base = s * RPT + q * ZROWS
            pltpu.sync_copy(zrow, acc_s.at[pl.ds(base, ZROWS)])
            pltpu.sync_copy(zdeg, deg_s.at[pl.ds(base, ZROWS)])

        pltpu.sync_copy(row_hbm.at[s], row_v)
        pltpu.sync_copy(col_hbm.at[s], col_v)

        ctile = jnp.full((16,), 0, jnp.int32) + c

        @pl.loop(0, NCHUNK)
        def _(j):
            @pl.loop(0, CHUNK // 16)
            def _(k):
                v = row_v[j, pl.ds(k * 16, 16)]
                rowx_v[j, pl.ds(k * 16, 16)] = v * 2 + ctile

        plsc.subcore_barrier()

        dlo = c * (NCHUNK // 2)

        @pl.loop(0, NCHUNK)
        def _(j):
            pltpu.sync_copy(x_hbm.at[rowx_v.at[j]], gbuf)
            pltpu.sync_copy(gbuf, acc_s.at[col_v.at[j]], add=True)

        @pl.loop(0, NCHUNK // 2)
        def _(j):
            pltpu.sync_copy(ones_v, deg_s.at[col_v.at[dlo + j]], add=True)

        plsc.subcore_barrier()

        out_base = c * ACC_N + s * RPT
        pltpu.sync_copy(acc_s.at[pl.ds(s * RPT, RPT)],
                        part_hbm.at[pl.ds(out_base, RPT)])
        pltpu.sync_copy(deg_s.at[pl.ds(s * RPT, RPT)],
                        degp_hbm.at[pl.ds(out_base, RPT)])

    return sc_kernel(x2, row_idx, col_idx)


def _tc_combine(x, part, degp, W_self, W_neigh, bias2d):
    R = 1000

    def body(x_ref, part_ref, degp_ref, ws_ref, wn_ref, b_ref, o_ref):
        a = jnp.concatenate([part_ref[0], part_ref[1]], axis=1)
        d = degp_ref[0] + degp_ref[1]
        dcol = jnp.maximum(d[:, 0:1], 1.0)
        agg = a / dcol
        o_ref[...] = (
            jnp.dot(x_ref[...], ws_ref[...], preferred_element_type=jnp.float32)
            + jnp.dot(agg, wn_ref[...], preferred_element_type=jnp.float32)
            + b_ref[...]
        )

    return pl.pallas_call(
        body,
        grid=(N // R,),
        in_specs=[
            pl.BlockSpec((R, D), lambda i: (i, 0)),
            pl.BlockSpec((NC, R, HD), lambda i: (0, i, 0)),
            pl.BlockSpec((NC, R, DEGW), lambda i: (0, i, 0)),
            pl.BlockSpec((D, D), lambda i: (0, 0)),
            pl.BlockSpec((D, D), lambda i: (0, 0)),
            pl.BlockSpec((1, D), lambda i: (0, 0)),
        ],
        out_specs=pl.BlockSpec((R, D), lambda i: (i, 0)),
        out_shape=jax.ShapeDtypeStruct((N, D), jnp.float32),
    )(x, part, degp, W_self, W_neigh, bias2d)


def kernel(x, edge_index, W_self, W_neigh, bias):
    ei = edge_index.astype(jnp.int32)
    row = ei[0].reshape(NS, NCHUNK, CHUNK)
    col = ei[1].reshape(NS, NCHUNK, CHUNK)
    x2 = x.reshape(2 * N, HD)
    part, degp = _sc_aggregate(x2, row, col)
    part = part.reshape(NC, ACC_N, HD)
    degp = degp.reshape(NC, ACC_N, DEGW)
    return _tc_combine(x, part, degp, W_self, W_neigh, bias.reshape(1, D))
